# baseline (speedup 1.0000x reference)
import functools

import jax
import jax.numpy as jnp
from jax import lax
from jax.experimental import pallas as pl
from jax.experimental.pallas import tpu as pltpu
from jax.experimental.pallas import tpu_sc as plsc

B, N_TOK, D = 8, 24576, 128
GRID = 49152
NC, NS, L = 2, 16, 16
NW = NC * NS
QPB = NW // B
RANGE = GRID // QPB
CHUNK = 128
NBLK = RANGE // CHUNK
ECHUNK = 64
NEBLK = RANGE // ECHUNK
WIN = 1024
NWIN = N_TOK // WIN
VPW = WIN // L
ELAG = 16
SENT = 0x7FFFFFFF


def _winner_scatter(tokens, idx):
  mesh = plsc.VectorSubcoreMesh(
      core_axis_name="c", subcore_axis_name="s",
      num_cores=NC, num_subcores=NS)

  @functools.partial(
      pl.kernel,
      out_type=jax.ShapeDtypeStruct((B, GRID, D), jnp.float32),
      mesh=mesh,
      compiler_params=pltpu.CompilerParams(needs_layout_passes=False),
      scratch_types=[
          pltpu.VMEM((2, WIN), jnp.int32),
          pltpu.VMEM((RANGE,), jnp.int32),
          pltpu.VMEM((L,), jnp.int32),
          pltpu.VMEM((WIN + L,), jnp.int32),
          pltpu.VMEM((RANGE + L,), jnp.int32),
          pltpu.VMEM((RANGE + L,), jnp.int32),
          pltpu.VMEM((NBLK, CHUNK), jnp.int32),
          pltpu.VMEM((NEBLK, ECHUNK), jnp.int32),
          pltpu.VMEM((ECHUNK, D), jnp.float32),
          pltpu.VMEM((2, CHUNK, D), jnp.float32),
          pltpu.SemaphoreType.DMA,
          pltpu.SemaphoreType.DMA,
          pltpu.SemaphoreType.DMA,
          pltpu.SemaphoreType.DMA,
      ],
  )
  def body(tokens_hbm, idx_hbm, out_hbm, idx_win, inv, bounce, entries,
           wtok, wcell, wcell2, ecell2, zblk, rows, sem_i, sem_z, sem_g,
           sem_s):
    wid = lax.axis_index("s") * NC + lax.axis_index("c")
    b = wid // QPB
    base = (wid % QPB) * RANGE

    iota = lax.iota(jnp.int32, L)
    zeros16 = jnp.zeros((L,), jnp.int32)
    zeros16f = jnp.zeros((L,), jnp.float32)
    neg16 = jnp.full((L,), -1, jnp.int32)
    shift_idx = jnp.minimum(iota + 1, L - 1)

    def init_inv(j, _):
      inv[pl.ds(j * L, L)] = neg16
      return 0
    lax.fori_loop(0, RANGE // L, init_inv, 0)

    def init_z(i, _):
      r = i // (D // L)
      c = (i % (D // L)) * L
      zblk[r, pl.ds(c, L)] = zeros16f
      return 0
    lax.fori_loop(0, ECHUNK * (D // L), init_z, 0)

    out_b = out_hbm.at[b]
    tok_b = tokens_hbm.at[b]

    pltpu.async_copy(idx_hbm.at[b, pl.ds(0, WIN)], idx_win.at[0], sem_i)
    for w in range(NWIN):
      if w + 1 < NWIN:
        pltpu.async_copy(idx_hbm.at[b, pl.ds((w + 1) * WIN, WIN)],
                         idx_win.at[(w + 1) % 2], sem_i)
      pltpu.make_async_copy(idx_hbm.at[b, pl.ds(w * WIN, WIN)],
                            idx_win.at[w % 2], sem_i).wait()

      def compact(k, ec, w=w):
        v = idx_win[w % 2, pl.ds(k * L, L)]
        local = v - base
        m = (local >= 0) & (local < RANGE)
        p = (w * WIN + k * L) + iota
        plsc.store_compressed(entries.at[pl.ds(ec, L)],
                              (local << 15) | p, mask=m)
        return ec + jnp.max(plsc.all_reduce_population_count(m))
      ec = lax.fori_loop(0, VPW, compact, jnp.int32(0))

      def flush(i, _):
        mvalid = (i * L + iota) < ec
        key = jnp.where(mvalid, entries[pl.ds(i * L, L)], SENT)
        skey, _ = plsc.sort_key_val(key, key)
        bounce[...] = skey
        snext = plsc.load_gather(bounce, [shift_idx])
        keep = ((skey >> 15) != (snext >> 15)) | (iota == L - 1)
        valid = skey != SENT
        plsc.store_scatter(inv, [skey >> 15], skey & 0x7FFF,
                           mask=keep & valid)
        return 0
      lax.fori_loop(0, (ec + L - 1) // L, flush, 0)

    def extract(j, cnt):
      v = inv[pl.ds(j * L, L)]
      m = v >= 0
      plsc.store_compressed(wtok.at[pl.ds(cnt, L)], v, mask=m)
      plsc.store_compressed(wcell.at[pl.ds(cnt, L)],
                            base + j * L + iota, mask=m)
      return cnt + jnp.max(plsc.all_reduce_population_count(m))
    cnt = lax.fori_loop(0, RANGE // L, extract, jnp.int32(0))

    nch = (cnt + CHUNK - 1) // CHUNK
    pend = nch * CHUNK
    ftok = plsc.load_gather(wtok, [zeros16])
    fcell = plsc.load_gather(wcell, [zeros16])
    start = (cnt // L) * L

    def pad(t, _):
      off = start + t * L

      @pl.when(off < pend)
      def _():
        m = (off + iota) >= cnt
        wtok[pl.ds(off, L)] = jnp.where(m, ftok, wtok[pl.ds(off, L)])
        wcell[pl.ds(off, L)] = jnp.where(m, fcell, wcell[pl.ds(off, L)])
      return 0
    lax.fori_loop(0, CHUNK // L, pad, 0)

    def repack(i, _):
      r = i // (CHUNK // L)
      c = (i % (CHUNK // L)) * L
      wcell2[r, pl.ds(c, L)] = wcell[pl.ds(i * L, L)]
      return 0
    lax.fori_loop(0, nch * (CHUNK // L), repack, 0)

    def g_copy(ci, buf):
      return pltpu.make_async_copy(
          tok_b.at[wtok.at[pl.ds(ci * CHUNK, CHUNK)]], rows.at[buf], sem_g)

    def s_copy(ci, buf):
      return pltpu.make_async_copy(
          rows.at[buf], out_b.at[wcell2.at[ci]], sem_s)

    @pl.when(nch > 0)
    def _():
      g_copy(0, 0).start()

    def extract_e(j, ce):
      v = inv[pl.ds(j * L, L)]
      m = v < 0
      plsc.store_compressed(wcell.at[pl.ds(ce, L)],
                            base + j * L + iota, mask=m)
      return ce + jnp.max(plsc.all_reduce_population_count(m))
    cnte = lax.fori_loop(0, RANGE // L, extract_e, jnp.int32(0))

    nech = (cnte + ECHUNK - 1) // ECHUNK
    epend = nech * ECHUNK
    fecell = plsc.load_gather(wcell, [zeros16])
    estart = (cnte // L) * L

    def pad_e(t, _):
      off = estart + t * L

      @pl.when(off < epend)
      def _():
        m = (off + iota) >= cnte
        wcell[pl.ds(off, L)] = jnp.where(m, fecell, wcell[pl.ds(off, L)])
      return 0
    lax.fori_loop(0, ECHUNK // L, pad_e, 0)

    def repack_e(i, _):
      r = i // (ECHUNK // L)
      c = (i % (ECHUNK // L)) * L
      ecell2[r, pl.ds(c, L)] = wcell[pl.ds(i * L, L)]
      return 0
    lax.fori_loop(0, nech * (ECHUNK // L), repack_e, 0)

    def e_issue(e):
      pltpu.async_copy(zblk, out_b.at[ecell2.at[e]], sem_z)

    def e_drain(e):
      pltpu.make_async_copy(zblk, out_b.at[ecell2.at[e]], sem_z).wait()

    def move(ci, carry):
      ei, ed = carry

      @pl.when(ci > 0)
      def _():
        s_copy(ci - 1, (ci - 1) % 2).wait()

      @pl.when(ci + 1 < nch)
      def _():
        g_copy(ci + 1, (ci + 1) % 2).start()
      g_copy(ci, ci % 2).wait()
      s_copy(ci, ci % 2).start()

      for _ in range(3):
        @pl.when(ei < nech)
        def _(ei=ei):
          e_issue(ei)
        ei = jnp.minimum(ei + 1, nech)

      def drain_more(edc):
        e_drain(edc)
        return edc + 1
      ed = lax.while_loop(lambda edc: edc < ei - ELAG, drain_more, ed)
      return ei, ed
    ei, ed = lax.fori_loop(0, nch, move, (jnp.int32(0), jnp.int32(0)))

    @pl.when(nch > 0)
    def _():
      s_copy(nch - 1, (nch - 1) % 2).wait()

    def tail_issue(carry):
      ei, ed = carry
      e_issue(ei)

      def drain_more(edc):
        e_drain(edc)
        return edc + 1
      ed = lax.while_loop(lambda edc: edc < ei + 1 - ELAG, drain_more, ed)
      return ei + 1, ed
    ei, ed = lax.while_loop(lambda c: c[0] < nech, tail_issue, (ei, ed))

    def drain_rest(edc):
      e_drain(edc)
      return edc + 1
    lax.while_loop(lambda edc: edc < nech, drain_rest, ed)

  return body(tokens, idx)


def kernel(tokens, idx, grid_size):
  del grid_size
  return _winner_scatter(tokens, idx.astype(jnp.int32))


# speedup vs baseline: 1.1629x; 1.1629x over previous
import functools

import jax
import jax.numpy as jnp
from jax import lax
from jax.experimental import pallas as pl
from jax.experimental.pallas import tpu as pltpu
from jax.experimental.pallas import tpu_sc as plsc

B, N_TOK, D = 8, 24576, 128
GRID = 49152
NC, NS, L = 2, 16, 16
NW = NC * NS
QPB = NW // B
RANGE = GRID // QPB
CHUNK = 128
NBLK = RANGE // CHUNK
ZGRP = 8
WIN = 2048
NWIN = N_TOK // WIN
VPW = WIN // L
SENT = 0x7FFFFFFF


def _winner_scatter(tokens, idx):
  mesh = plsc.VectorSubcoreMesh(
      core_axis_name="c", subcore_axis_name="s",
      num_cores=NC, num_subcores=NS)

  @functools.partial(
      pl.kernel,
      out_type=jax.ShapeDtypeStruct((B, GRID, D), jnp.float32),
      mesh=mesh,
      compiler_params=pltpu.CompilerParams(needs_layout_passes=False),
      scratch_types=[
          pltpu.VMEM((2, WIN), jnp.int32),
          pltpu.VMEM((RANGE,), jnp.int32),
          pltpu.VMEM((L,), jnp.int32),
          pltpu.VMEM((WIN + L,), jnp.int32),
          pltpu.VMEM((RANGE + L,), jnp.int32),
          pltpu.VMEM((RANGE + L,), jnp.int32),
          pltpu.VMEM((NBLK, CHUNK), jnp.int32),
          pltpu.VMEM((4, CHUNK, D), jnp.float32),
          pltpu.SemaphoreType.DMA,
          pltpu.SemaphoreType.DMA,
          pltpu.SemaphoreType.DMA,
          pltpu.SemaphoreType.DMA,
      ],
  )
  def body(tokens_hbm, idx_hbm, out_hbm, idx_win, inv, bounce, entries,
           wtok, wcell, wcell2, rows, sem_i, sem_z, sem_g, sem_s):
    wid = lax.axis_index("s") * NC + lax.axis_index("c")
    b = wid // QPB
    base = (wid % QPB) * RANGE

    iota = lax.iota(jnp.int32, L)
    zeros16f = jnp.zeros((L,), jnp.float32)
    neg16 = jnp.full((L,), -1, jnp.int32)
    shift_idx = jnp.minimum(iota + 1, L - 1)

    def init_inv(j, _):
      inv[pl.ds(j * L, L)] = neg16
      return 0
    lax.fori_loop(0, RANGE // L, init_inv, 0)

    def init_z(i, _):
      r = i // (D // L)
      c = (i % (D // L)) * L
      rows[0, r, pl.ds(c, L)] = zeros16f
      return 0
    lax.fori_loop(0, CHUNK * (D // L), init_z, 0)

    out_b = out_hbm.at[b]
    tok_b = tokens_hbm.at[b]

    def zero_start(g):
      for t in range(ZGRP):
        blk = g * ZGRP + t
        pltpu.async_copy(
            rows.at[0], out_b.at[pl.ds(base + blk * CHUNK, CHUNK)], sem_z)

    def zero_drain(g):
      for t in range(ZGRP):
        blk = g * ZGRP + t
        pltpu.make_async_copy(
            rows.at[0], out_b.at[pl.ds(base + blk * CHUNK, CHUNK)], sem_z).wait()

    pltpu.async_copy(idx_hbm.at[b, pl.ds(0, WIN)], idx_win.at[0], sem_i)
    for w in range(NWIN):
      if w + 1 < NWIN:
        pltpu.async_copy(idx_hbm.at[b, pl.ds((w + 1) * WIN, WIN)],
                         idx_win.at[(w + 1) % 2], sem_i)
      pltpu.make_async_copy(idx_hbm.at[b, pl.ds(w * WIN, WIN)],
                            idx_win.at[w % 2], sem_i).wait()
      zero_start(w)

      def compact(k, ec, w=w):
        v = idx_win[w % 2, pl.ds(k * L, L)]
        local = v - base
        m = (local >= 0) & (local < RANGE)
        p = (w * WIN + k * L) + iota
        plsc.store_compressed(entries.at[pl.ds(ec, L)],
                              (local << 15) | p, mask=m)
        return ec + jnp.max(plsc.all_reduce_population_count(m))
      ec = lax.fori_loop(0, VPW, compact, jnp.int32(0))

      def flush(i, _):
        mvalid = (i * L + iota) < ec
        key = jnp.where(mvalid, entries[pl.ds(i * L, L)], SENT)
        skey, _ = plsc.sort_key_val(key, key)
        bounce[...] = skey
        snext = plsc.load_gather(bounce, [shift_idx])
        keep = ((skey >> 15) != (snext >> 15)) | (iota == L - 1)
        valid = skey != SENT
        plsc.store_scatter(inv, [skey >> 15], skey & 0x7FFF,
                           mask=keep & valid)
        return 0
      lax.fori_loop(0, (ec + L - 1) // L, flush, 0)
      if w >= 2:
        zero_drain(w - 2)

    def extract(j, cnt):
      v = inv[pl.ds(j * L, L)]
      m = v >= 0
      plsc.store_compressed(wtok.at[pl.ds(cnt, L)], v, mask=m)
      plsc.store_compressed(wcell.at[pl.ds(cnt, L)],
                            base + j * L + iota, mask=m)
      return cnt + jnp.max(plsc.all_reduce_population_count(m))
    cnt = lax.fori_loop(0, RANGE // L, extract, jnp.int32(0))
    zero_drain(NWIN - 2)
    zero_drain(NWIN - 1)

    @pl.when(cnt > 0)
    def _():
      nch = (cnt + CHUNK - 1) // CHUNK
      pend = nch * CHUNK
      ftok = plsc.load_gather(wtok, [jnp.zeros((L,), jnp.int32)])
      fcell = plsc.load_gather(wcell, [jnp.zeros((L,), jnp.int32)])
      start = (cnt // L) * L

      def pad(t, _):
        off = start + t * L

        @pl.when(off < pend)
        def _():
          m = (off + iota) >= cnt
          wtok[pl.ds(off, L)] = jnp.where(m, ftok, wtok[pl.ds(off, L)])
          wcell[pl.ds(off, L)] = jnp.where(m, fcell, wcell[pl.ds(off, L)])
        return 0
      lax.fori_loop(0, CHUNK // L, pad, 0)

      def repack(i, _):
        r = i // (CHUNK // L)
        c = (i % (CHUNK // L)) * L
        wcell2[r, pl.ds(c, L)] = wcell[pl.ds(i * L, L)]
        return 0
      lax.fori_loop(0, nch * (CHUNK // L), repack, 0)

      def g_copy(ci, buf):
        return pltpu.make_async_copy(
            tok_b.at[wtok.at[pl.ds(ci * CHUNK, CHUNK)]], rows.at[buf], sem_g)

      def s_copy(ci, buf):
        return pltpu.make_async_copy(
            rows.at[buf], out_b.at[wcell2.at[ci]], sem_s)

      g_copy(0, 0).start()

      @pl.when(nch > 1)
      def _():
        g_copy(1, 1).start()

      def move(ci, _):
        @pl.when(ci > 1)
        def _():
          s_copy(ci - 2, (ci - 2) % 4).wait()

        @pl.when(ci + 2 < nch)
        def _():
          g_copy(ci + 2, (ci + 2) % 4).start()
        g_copy(ci, ci % 4).wait()
        s_copy(ci, ci % 4).start()
        return 0
      lax.fori_loop(0, nch, move, 0)

      @pl.when(nch > 1)
      def _():
        s_copy(nch - 2, (nch - 2) % 4).wait()
      s_copy(nch - 1, (nch - 1) % 4).wait()

  return body(tokens, idx)


def kernel(tokens, idx, grid_size):
  del grid_size
  return _winner_scatter(tokens, idx.astype(jnp.int32))


# speedup vs baseline: 1.1674x; 1.0039x over previous
import functools

import jax
import jax.numpy as jnp
from jax import lax
from jax.experimental import pallas as pl
from jax.experimental.pallas import tpu as pltpu
from jax.experimental.pallas import tpu_sc as plsc

B, N_TOK, D = 8, 24576, 128
GRID = 49152
NC, NS, L = 2, 16, 16
NW = NC * NS
QPB = NW // B
RANGE = GRID // QPB
CHUNK = 128
NBLK = RANGE // CHUNK
ZGRP = 8
WIN = 2048
NWIN = N_TOK // WIN
VPW = WIN // L
SENT = 0x7FFFFFFF


def _winner_scatter(tokens, idx):
  mesh = plsc.VectorSubcoreMesh(
      core_axis_name="c", subcore_axis_name="s",
      num_cores=NC, num_subcores=NS)

  @functools.partial(
      pl.kernel,
      out_type=jax.ShapeDtypeStruct((B, GRID, D), jnp.float32),
      mesh=mesh,
      compiler_params=pltpu.CompilerParams(needs_layout_passes=False),
      scratch_types=[
          pltpu.VMEM((2, WIN), jnp.int32),
          pltpu.VMEM((RANGE,), jnp.int32),
          pltpu.VMEM((L,), jnp.int32),
          pltpu.VMEM((WIN + L,), jnp.int32),
          pltpu.VMEM((RANGE + L,), jnp.int32),
          pltpu.VMEM((RANGE + L,), jnp.int32),
          pltpu.VMEM((NBLK, CHUNK), jnp.int32),
          pltpu.VMEM((4, CHUNK, D), jnp.float32),
          pltpu.SemaphoreType.DMA,
          pltpu.SemaphoreType.DMA,
          pltpu.SemaphoreType.DMA,
          pltpu.SemaphoreType.DMA,
      ],
  )
  def body(tokens_hbm, idx_hbm, out_hbm, idx_win, inv, bounce, entries,
           wtok, wcell, wcell2, rows, sem_i, sem_z, sem_g, sem_s):
    wid = lax.axis_index("s") * NC + lax.axis_index("c")
    b = wid // QPB
    base = (wid % QPB) * RANGE

    iota = lax.iota(jnp.int32, L)
    zeros16f = jnp.zeros((L,), jnp.float32)
    neg16 = jnp.full((L,), -1, jnp.int32)
    shift_idx = jnp.minimum(iota + 1, L - 1)

    pltpu.async_copy(idx_hbm.at[b, pl.ds(0, WIN)], idx_win.at[0], sem_i)

    def init_inv(j, _):
      inv[pl.ds(j * L, L)] = neg16
      return 0
    lax.fori_loop(0, RANGE // L, init_inv, 0)

    def init_z(i, _):
      r = i // (D // L)
      c = (i % (D // L)) * L
      rows[0, r, pl.ds(c, L)] = zeros16f
      return 0
    lax.fori_loop(0, CHUNK * (D // L), init_z, 0)

    out_b = out_hbm.at[b]
    tok_b = tokens_hbm.at[b]

    def zero_start(g):
      for t in range(ZGRP):
        blk = g * ZGRP + t
        pltpu.async_copy(
            rows.at[0], out_b.at[pl.ds(base + blk * CHUNK, CHUNK)], sem_z)

    def zero_drain(g):
      for t in range(ZGRP):
        blk = g * ZGRP + t
        pltpu.make_async_copy(
            rows.at[0], out_b.at[pl.ds(base + blk * CHUNK, CHUNK)], sem_z).wait()

    for w in range(NWIN):
      if w + 1 < NWIN:
        pltpu.async_copy(idx_hbm.at[b, pl.ds((w + 1) * WIN, WIN)],
                         idx_win.at[(w + 1) % 2], sem_i)
      pltpu.make_async_copy(idx_hbm.at[b, pl.ds(w * WIN, WIN)],
                            idx_win.at[w % 2], sem_i).wait()
      zero_start(w)

      def compact(k, ec, w=w):
        v = idx_win[w % 2, pl.ds(k * L, L)]
        local = v - base
        m = (local >= 0) & (local < RANGE)
        p = (w * WIN + k * L) + iota
        plsc.store_compressed(entries.at[pl.ds(ec, L)],
                              (local << 15) | p, mask=m)
        return ec + jnp.max(plsc.all_reduce_population_count(m))
      ec = lax.fori_loop(0, VPW, compact, jnp.int32(0))

      def flush(i, _):
        mvalid = (i * L + iota) < ec
        key = jnp.where(mvalid, entries[pl.ds(i * L, L)], SENT)
        skey, _ = plsc.sort_key_val(key, key)
        bounce[...] = skey
        snext = plsc.load_gather(bounce, [shift_idx])
        keep = ((skey >> 15) != (snext >> 15)) | (iota == L - 1)
        valid = skey != SENT
        plsc.store_scatter(inv, [skey >> 15], skey & 0x7FFF,
                           mask=keep & valid)
        return 0
      lax.fori_loop(0, (ec + L - 1) // L, flush, 0)
      if w >= 3:
        zero_drain(w - 3)

    def extract(j, cnt):
      v = inv[pl.ds(j * L, L)]
      m = v >= 0
      plsc.store_compressed(wtok.at[pl.ds(cnt, L)], v, mask=m)
      plsc.store_compressed(wcell.at[pl.ds(cnt, L)],
                            base + j * L + iota, mask=m)
      return cnt + jnp.max(plsc.all_reduce_population_count(m))
    cnt = lax.fori_loop(0, RANGE // L, extract, jnp.int32(0))
    zero_drain(NWIN - 3)
    zero_drain(NWIN - 2)
    zero_drain(NWIN - 1)

    @pl.when(cnt > 0)
    def _():
      nch = (cnt + CHUNK - 1) // CHUNK
      pend = nch * CHUNK
      ftok = plsc.load_gather(wtok, [jnp.zeros((L,), jnp.int32)])
      fcell = plsc.load_gather(wcell, [jnp.zeros((L,), jnp.int32)])
      start = (cnt // L) * L

      def pad(t, _):
        off = start + t * L

        @pl.when(off < pend)
        def _():
          m = (off + iota) >= cnt
          wtok[pl.ds(off, L)] = jnp.where(m, ftok, wtok[pl.ds(off, L)])
          wcell[pl.ds(off, L)] = jnp.where(m, fcell, wcell[pl.ds(off, L)])
        return 0
      lax.fori_loop(0, CHUNK // L, pad, 0)

      def repack(i, _):
        r = i // (CHUNK // L)
        c = (i % (CHUNK // L)) * L
        wcell2[r, pl.ds(c, L)] = wcell[pl.ds(i * L, L)]
        return 0
      lax.fori_loop(0, nch * (CHUNK // L), repack, 0)

      def g_copy(ci, buf):
        return pltpu.make_async_copy(
            tok_b.at[wtok.at[pl.ds(ci * CHUNK, CHUNK)]], rows.at[buf], sem_g)

      def s_copy(ci, buf):
        return pltpu.make_async_copy(
            rows.at[buf], out_b.at[wcell2.at[ci]], sem_s)

      g_copy(0, 0).start()

      @pl.when(nch > 1)
      def _():
        g_copy(1, 1).start()

      def move(ci, _):
        @pl.when(ci > 1)
        def _():
          s_copy(ci - 2, (ci - 2) % 4).wait()

        @pl.when(ci + 2 < nch)
        def _():
          g_copy(ci + 2, (ci + 2) % 4).start()
        g_copy(ci, ci % 4).wait()
        s_copy(ci, ci % 4).start()
        return 0
      lax.fori_loop(0, nch, move, 0)

      @pl.when(nch > 1)
      def _():
        s_copy(nch - 2, (nch - 2) % 4).wait()
      s_copy(nch - 1, (nch - 1) % 4).wait()

  return body(tokens, idx)


def kernel(tokens, idx, grid_size):
  del grid_size
  return _winner_scatter(tokens, idx.astype(jnp.int32))
